# baseline (speedup 1.0000x reference)
import functools

import jax
import jax.numpy as jnp
from jax import lax
from jax.experimental import pallas as pl
from jax.experimental.pallas import tpu as pltpu
from jax.experimental.pallas import tpu_sc as plsc

HIDDEN = 64
BATCH = 16384

_NC = 2
_NS = 16
_NW = _NC * _NS
_BPW = BATCH // _NW
_CHUNK = 128
_NCHUNK = _BPW // _CHUNK


def _sc_gather(table, ids3):
    mesh = plsc.VectorSubcoreMesh(core_axis_name="c", subcore_axis_name="s")

    @functools.partial(
        pl.kernel,
        out_type=jax.ShapeDtypeStruct((_NW, _NCHUNK, _CHUNK, HIDDEN), jnp.float32),
        mesh=mesh,
        scratch_types=[
            pltpu.VMEM((_NCHUNK, _CHUNK), jnp.int32),
            pltpu.VMEM((_NCHUNK, _CHUNK, HIDDEN), jnp.float32),
            pltpu.SemaphoreType.DMA,
        ],
        compiler_params=pltpu.CompilerParams(use_tc_tiling_on_sc=False),
    )
    def k(table_hbm, ids_hbm, out_hbm, idx_v, rows_v, sem):
        wid = lax.axis_index("s") * _NC + lax.axis_index("c")
        pltpu.sync_copy(ids_hbm.at[wid], idx_v)
        copies = [
            pltpu.async_copy(table_hbm.at[idx_v.at[j]], rows_v.at[j], sem)
            for j in range(_NCHUNK)
        ]
        for c in copies:
            c.wait()
        pltpu.sync_copy(rows_v, out_hbm.at[wid])

    return k(table, ids3)


def _tc_body(x_ref, pe_ref, a_ref, p_ref, ae_ref, pt_ref, w_ref, b_ref,
             g_ref, bt_ref, o_ref):
    h = x_ref[...] + pe_ref[...]
    a = a_ref[...]
    for k in range(3):
        h += jnp.where(a == k, ae_ref[k, :][None, :], 0.0)
    p = p_ref[...]
    for k in range(10):
        h += jnp.where(p == k, pt_ref[k, :][None, :], 0.0)
    hw = lax.dot_general(h, w_ref[...], (((1,), (1,)), ((), ())),
                         preferred_element_type=jnp.float32) + b_ref[...]
    mean = jnp.mean(hw, axis=1, keepdims=True)
    cen = hw - mean
    var = jnp.mean(cen * cen, axis=1, keepdims=True)
    o_ref[...] = cen * lax.rsqrt(var + 1e-5) * g_ref[...] + bt_ref[...]


def _tc_fused(x, pe, a2, p2, action_emb, pos_emb, W, b2, g2, bt2, blk=2048):
    grid = BATCH // blk
    return pl.pallas_call(
        _tc_body,
        grid=(grid,),
        in_specs=[
            pl.BlockSpec((blk, HIDDEN), lambda i: (i, 0)),
            pl.BlockSpec((blk, HIDDEN), lambda i: (i, 0)),
            pl.BlockSpec((blk, 1), lambda i: (i, 0)),
            pl.BlockSpec((blk, 1), lambda i: (i, 0)),
            pl.BlockSpec((3, HIDDEN), lambda i: (0, 0)),
            pl.BlockSpec((10, HIDDEN), lambda i: (0, 0)),
            pl.BlockSpec((HIDDEN, HIDDEN), lambda i: (0, 0)),
            pl.BlockSpec((1, HIDDEN), lambda i: (0, 0)),
            pl.BlockSpec((1, HIDDEN), lambda i: (0, 0)),
            pl.BlockSpec((1, HIDDEN), lambda i: (0, 0)),
        ],
        out_specs=pl.BlockSpec((blk, HIDDEN), lambda i: (i, 0)),
        out_shape=jax.ShapeDtypeStruct((BATCH, HIDDEN), jnp.float32),
    )(x, pe, a2, p2, action_emb, pos_emb, W, b2, g2, bt2)


def kernel(x, player_ids, actions, positions, player_emb, action_emb,
           pos_emb, W, b, gamma, beta):
    ids3 = player_ids.astype(jnp.int32).reshape(_NW, _NCHUNK, _CHUNK)
    pe = _sc_gather(player_emb, ids3).reshape(BATCH, HIDDEN)
    a2 = actions.astype(jnp.int32).reshape(BATCH, 1)
    p2 = positions.astype(jnp.int32).reshape(BATCH, 1)
    out = _tc_fused(x, pe, a2, p2, action_emb, pos_emb, W,
                    b.reshape(1, HIDDEN), gamma.reshape(1, HIDDEN),
                    beta.reshape(1, HIDDEN))
    return out.reshape(BATCH, 1, HIDDEN)
